# baseline (speedup 1.0000x reference)
import functools

import jax
import jax.numpy as jnp
from jax import lax
from jax.experimental import pallas as pl
from jax.experimental.pallas import tpu as pltpu
from jax.experimental.pallas import tpu_sc as plsc

_COMPLEMENT = (0, 1, 2, 3, 4, 5, 6, 10, 9, 8, 7, 11)


def _table_body(reps, w_ref, out_ref):
    w = w_ref[...]
    d = w.shape[1]
    rc = jnp.concatenate([w_ref[c:c + 1, :] for c in _COMPLEMENT], axis=0)
    ri = lax.broadcasted_iota(jnp.int32, (d, d), 0)
    ci = lax.broadcasted_iota(jnp.int32, (d, d), 1)
    rev = jnp.where(ri + ci == d - 1, 1.0, 0.0).astype(w.dtype)
    fused = jnp.concatenate(
        [w, jnp.dot(rc, rev, preferred_element_type=jnp.float32)], axis=1
    )
    v = w.shape[0]
    for r in range(reps):
        out_ref[pl.ds(r * v, v), :] = fused


def _build_table(W, reps):
    v, d = W.shape
    return pl.pallas_call(
        functools.partial(_table_body, reps),
        out_shape=jax.ShapeDtypeStruct((reps * v, 2 * d), W.dtype),
    )(W)


@functools.lru_cache(maxsize=None)
def _make_gather(n, v, d2):
    info = plsc.get_sparse_core_info()
    nc, ns = info.num_cores, info.num_subcores
    nw = nc * ns
    per_w = n // nw
    assert per_w * nw == n
    chunk = 32
    nbuf = 4
    nch = per_w // chunk
    assert nch * chunk == per_w
    ngrp = nch // nbuf
    assert ngrp * nbuf == nch
    mesh = plsc.VectorSubcoreMesh(core_axis_name="c", subcore_axis_name="s")

    @functools.partial(
        pl.kernel,
        mesh=mesh,
        out_type=jax.ShapeDtypeStruct((n, d2), jnp.float32),
        scratch_types=[
            pltpu.VMEM((per_w,), jnp.int32),
        ] + [pltpu.VMEM((chunk, d2), jnp.float32)] * nbuf
          + [pltpu.SemaphoreType.DMA] * (2 * nbuf),
    )
    def gk(table_hbm, idx_hbm, out_hbm, idx_v, *bufs_sems):
        bufs = bufs_sems[:nbuf]
        sgs = bufs_sems[nbuf:2 * nbuf]
        sws = bufs_sems[2 * nbuf:]
        wid = lax.axis_index("s") * nc + lax.axis_index("c")
        base = wid * per_w
        pltpu.sync_copy(idx_hbm.at[pl.ds(base, per_w)], idx_v)

        off = wid * v

        def obody(t, carry):
            sl = pl.ds(t * 16, 16)
            idx_v[sl] = idx_v[sl] + off
            return carry

        lax.fori_loop(0, per_w // 16, obody, 0, unroll=8)

        def g_start(j, buf, sem):
            pltpu.async_copy(
                table_hbm.at[idx_v.at[pl.ds(j * chunk, chunk)]], buf, sem
            )

        def g_wait(buf, sem):
            pltpu.make_async_copy(
                table_hbm.at[idx_v.at[pl.ds(0, chunk)]], buf, sem
            ).wait()

        def w_start(j, buf, sem):
            pltpu.async_copy(buf, out_hbm.at[pl.ds(base + j * chunk, chunk)], sem)

        def w_wait(buf, sem):
            pltpu.make_async_copy(buf, out_hbm.at[pl.ds(base, chunk)], sem).wait()

        for b in range(nbuf):
            g_start(b, bufs[b], sgs[b])

        def body(i, carry):
            j0 = nbuf * i
            for b in range(nbuf):
                g_wait(bufs[b], sgs[b])
                w_start(j0 + b, bufs[b], sws[b])

            @pl.when(i + 1 < ngrp)
            def _():
                for b in range(nbuf):
                    w_wait(bufs[b], sws[b])
                    g_start(j0 + nbuf + b, bufs[b], sgs[b])

            return carry

        lax.fori_loop(0, ngrp, body, 0)
        for b in range(nbuf):
            w_wait(bufs[b], sws[b])

    return gk


def kernel(input_ids, W):
    b, l = input_ids.shape
    v, d = W.shape
    info = plsc.get_sparse_core_info()
    nw = info.num_cores * info.num_subcores
    table = _build_table(W, nw)
    ids = input_ids.reshape(b * l)
    out = _make_gather(b * l, v, 2 * d)(table, ids)
    return out.reshape(b, l, 2 * d)
